# baseline (device time: 114356 ns/iter reference)
import jax
import jax.numpy as jnp
from jax import lax
from jax.experimental import pallas as pl
from jax.experimental.pallas import tpu as pltpu

N_DEV = 4
E = 32
E_LOCAL = 8
CAP = 51
CAP_PAD = 64
N_TOK = 2048
D = 512
H = 1024
M = N_TOK // N_DEV


def _matmul_body(xg_ref, w_ref, out_ref):
    out_ref[0] = jnp.dot(
        xg_ref[0], w_ref[0], preferred_element_type=jnp.float32
    )


def _rs_body(p_ref, out_ref, comm_ref, send_sems, recv_sems):
    my = lax.axis_index("i")
    left = jnp.mod(my - 1, N_DEV)
    right = jnp.mod(my + 1, N_DEV)

    barrier_sem = pltpu.get_barrier_semaphore()
    for nbr in (left, right):
        pl.semaphore_signal(
            barrier_sem, inc=1,
            device_id=(nbr,), device_id_type=pl.DeviceIdType.MESH,
        )
    pl.semaphore_wait(barrier_sem, 2)

    for s in range(N_DEV - 1):
        c_send = jnp.mod(my - 1 - s, N_DEV)
        if s == 0:
            src = p_ref.at[pl.ds(c_send * M, M), :]
        else:
            src = comm_ref.at[s - 1]
        rdma = pltpu.make_async_remote_copy(
            src_ref=src,
            dst_ref=comm_ref.at[s],
            send_sem=send_sems.at[s],
            recv_sem=recv_sems.at[s],
            device_id=(right,),
            device_id_type=pl.DeviceIdType.MESH,
        )
        rdma.start()
        rdma.wait()

        c_recv = jnp.mod(my - 2 - s, N_DEV)
        if s < N_DEV - 2:
            comm_ref[s] = comm_ref[s] + p_ref[pl.ds(c_recv * M, M), :]
        else:
            out_ref[:, :] = comm_ref[s] + p_ref[pl.ds(c_recv * M, M), :]


def kernel(x, router_W, route_idx, expert_W):
    del router_W
    my = lax.axis_index("i")

    e = route_idx[:, 0].astype(jnp.int32)
    onehot = (e[:, None] == jnp.arange(E, dtype=jnp.int32)[None, :])
    cum = jnp.cumsum(onehot.astype(jnp.int32), axis=0)
    pos = jnp.take_along_axis(cum, e[:, None], axis=1)[:, 0] - 1
    keep = pos < CAP
    el = e - E_LOCAL * my
    mine = (el >= 0) & (el < E_LOCAL) & keep
    slot = jnp.where(mine, el * CAP_PAD + pos, E_LOCAL * CAP_PAD)
    tokens = jnp.arange(N_TOK, dtype=jnp.int32)
    token_for_slot = (
        jnp.full((E_LOCAL * CAP_PAD,), N_TOK, jnp.int32)
        .at[slot].set(tokens, mode="drop")
    )
    x_pad = jnp.concatenate([x, jnp.zeros((1, D), x.dtype)], axis=0)
    xg = x_pad[token_for_slot].reshape(E_LOCAL, CAP_PAD, D)

    yg = pl.pallas_call(
        _matmul_body,
        grid=(E_LOCAL,),
        in_specs=[
            pl.BlockSpec((1, CAP_PAD, D), lambda i: (i, 0, 0)),
            pl.BlockSpec((1, D, H), lambda i: (i, 0, 0)),
        ],
        out_specs=pl.BlockSpec((1, CAP_PAD, H), lambda i: (i, 0, 0)),
        out_shape=jax.ShapeDtypeStruct((E_LOCAL, CAP_PAD, H), jnp.float32),
    )(xg, expert_W)

    partial = (
        jnp.zeros((N_TOK, H), jnp.float32)
        .at[token_for_slot].set(yg.reshape(E_LOCAL * CAP_PAD, H), mode="drop")
    )

    out = pl.pallas_call(
        _rs_body,
        out_shape=jax.ShapeDtypeStruct((M, H), jnp.float32),
        in_specs=[pl.BlockSpec(memory_space=pltpu.VMEM)],
        out_specs=pl.BlockSpec(memory_space=pltpu.VMEM),
        scratch_shapes=[
            pltpu.VMEM((N_DEV - 1, M, H), jnp.float32),
            pltpu.SemaphoreType.DMA((N_DEV - 1,)),
            pltpu.SemaphoreType.DMA((N_DEV - 1,)),
        ],
        compiler_params=pltpu.CompilerParams(collective_id=0),
    )(partial)
    return out


# device time: 101697 ns/iter; 1.1245x vs baseline; 1.1245x over previous
import jax
import jax.numpy as jnp
from jax import lax
from jax.experimental import pallas as pl
from jax.experimental.pallas import tpu as pltpu

N_DEV = 4
E = 32
E_LOCAL = 8
CAP = 51
CAP_PAD = 64
N_TOK = 2048
D = 512
H = 1024
M = N_TOK // N_DEV
S = E_LOCAL * CAP_PAD


def _fused_body(slot_ref, slotc_ref, x_ref, w_ref, out_ref, yg_ref, snd_ref,
                comm_ref, send_sems, recv_sems):
    my = lax.axis_index("i")
    left = jnp.mod(my - 1, N_DEV)
    right = jnp.mod(my + 1, N_DEV)

    barrier_sem = pltpu.get_barrier_semaphore()
    for nbr in (left, right):
        pl.semaphore_signal(
            barrier_sem, inc=1,
            device_id=(nbr,), device_id_type=pl.DeviceIdType.MESH,
        )
    pl.semaphore_wait(barrier_sem, 2)

    slot_t = slot_ref[:, :]
    iota_s = lax.broadcasted_iota(jnp.int32, (S, N_TOK), 0)
    P = (iota_s == slot_t).astype(jnp.float32)
    xg = jnp.dot(P, x_ref[:, :], preferred_element_type=jnp.float32)

    for e in range(E_LOCAL):
        yg_ref[e * CAP_PAD:(e + 1) * CAP_PAD, :] = jnp.dot(
            xg[e * CAP_PAD:(e + 1) * CAP_PAD, :], w_ref[e],
            preferred_element_type=jnp.float32,
        )
    yg = yg_ref[:, :]

    def partial_chunk(c):
        slot_c = slotc_ref[pl.ds(c * M, M), :]
        iota_r = lax.broadcasted_iota(jnp.int32, (M, S), 1)
        pct = (iota_r == slot_c).astype(jnp.float32)
        return jnp.dot(pct, yg, preferred_element_type=jnp.float32)

    snd_ref[:, :] = partial_chunk(jnp.mod(my - 1, N_DEV))
    rdma0 = pltpu.make_async_remote_copy(
        src_ref=snd_ref, dst_ref=comm_ref.at[0],
        send_sem=send_sems.at[0], recv_sem=recv_sems.at[0],
        device_id=(right,), device_id_type=pl.DeviceIdType.MESH,
    )
    rdma0.start()
    a1 = partial_chunk(jnp.mod(my - 2, N_DEV))
    rdma0.wait()

    comm_ref[0] = comm_ref[0] + a1
    rdma1 = pltpu.make_async_remote_copy(
        src_ref=comm_ref.at[0], dst_ref=comm_ref.at[1],
        send_sem=send_sems.at[1], recv_sem=recv_sems.at[1],
        device_id=(right,), device_id_type=pl.DeviceIdType.MESH,
    )
    rdma1.start()
    a2 = partial_chunk(jnp.mod(my - 3, N_DEV))
    rdma1.wait()

    comm_ref[1] = comm_ref[1] + a2
    rdma2 = pltpu.make_async_remote_copy(
        src_ref=comm_ref.at[1], dst_ref=comm_ref.at[2],
        send_sem=send_sems.at[2], recv_sem=recv_sems.at[2],
        device_id=(right,), device_id_type=pl.DeviceIdType.MESH,
    )
    rdma2.start()
    a3 = partial_chunk(my)
    rdma2.wait()

    out_ref[:, :] = comm_ref[2] + a3


def kernel(x, router_W, route_idx, expert_W):
    del router_W
    my = lax.axis_index("i")

    e = route_idx[:, 0].astype(jnp.int32)
    onehot = (e[:, None] == jnp.arange(E, dtype=jnp.int32)[None, :])
    cum = jnp.cumsum(onehot.astype(jnp.int32), axis=0)
    pos = jnp.sum(jnp.where(onehot, cum, 0), axis=1) - 1
    keep = pos < CAP
    el = e - E_LOCAL * my
    mine = (el >= 0) & (el < E_LOCAL) & keep
    slot = jnp.where(mine, el * CAP_PAD + pos, S).astype(jnp.int32)

    return pl.pallas_call(
        _fused_body,
        out_shape=jax.ShapeDtypeStruct((M, H), jnp.float32),
        in_specs=[
            pl.BlockSpec(memory_space=pltpu.VMEM),
            pl.BlockSpec(memory_space=pltpu.VMEM),
            pl.BlockSpec(memory_space=pltpu.VMEM),
            pl.BlockSpec(memory_space=pltpu.VMEM),
        ],
        out_specs=pl.BlockSpec(memory_space=pltpu.VMEM),
        scratch_shapes=[
            pltpu.VMEM((S, H), jnp.float32),
            pltpu.VMEM((M, H), jnp.float32),
            pltpu.VMEM((N_DEV - 1, M, H), jnp.float32),
            pltpu.SemaphoreType.DMA((N_DEV - 1,)),
            pltpu.SemaphoreType.DMA((N_DEV - 1,)),
        ],
        compiler_params=pltpu.CompilerParams(collective_id=0),
    )(slot[None, :], slot[:, None], x, expert_W)


# device time: 76441 ns/iter; 1.4960x vs baseline; 1.3304x over previous
import jax
import jax.numpy as jnp
from jax import lax
from jax.experimental import pallas as pl
from jax.experimental.pallas import tpu as pltpu

N_DEV = 4
E = 32
E_LOCAL = 8
CAP = 51
CAP_PAD = 64
N_TOK = 2048
D = 512
H = 1024
M = N_TOK // N_DEV
S = E_LOCAL * CAP_PAD


def _fused_body(slot_ref, slotc_ref, x_ref, w_ref, out_ref, yg_ref, snd_ref,
                comm_ref, send_sems, recv_sems):
    my = lax.axis_index("i")
    left = jnp.mod(my - 1, N_DEV)
    right = jnp.mod(my + 1, N_DEV)
    diag = jnp.mod(my + 2, N_DEV)

    barrier_sem = pltpu.get_barrier_semaphore()
    for nbr in (left, right, diag):
        pl.semaphore_signal(
            barrier_sem, inc=1,
            device_id=(nbr,), device_id_type=pl.DeviceIdType.MESH,
        )
    pl.semaphore_wait(barrier_sem, 3)

    slot_t = slot_ref[:, :]
    iota_s = lax.broadcasted_iota(jnp.int32, (S, N_TOK), 0)
    P = (iota_s == slot_t).astype(jnp.float32)
    xg = jnp.dot(P, x_ref[:, :], preferred_element_type=jnp.float32)

    for e in range(E_LOCAL):
        yg_ref[e * CAP_PAD:(e + 1) * CAP_PAD, :] = jnp.dot(
            xg[e * CAP_PAD:(e + 1) * CAP_PAD, :], w_ref[e],
            preferred_element_type=jnp.float32,
        )
    yg = yg_ref[:, :]

    def partial_chunk(c):
        slot_c = slotc_ref[pl.ds(c * M, M), :]
        iota_r = lax.broadcasted_iota(jnp.int32, (M, S), 1)
        pct = (iota_r == slot_c).astype(jnp.float32)
        return jnp.dot(pct, yg, preferred_element_type=jnp.float32)

    dsts = (diag, right, left)
    slots = (1, 0, 2)
    rdmas = []
    for (dst, sl) in zip(dsts, slots):
        snd_ref[sl] = partial_chunk(dst)
        r = pltpu.make_async_remote_copy(
            src_ref=snd_ref.at[sl], dst_ref=comm_ref.at[sl],
            send_sem=send_sems.at[sl], recv_sem=recv_sems.at[sl],
            device_id=(dst,), device_id_type=pl.DeviceIdType.MESH,
        )
        r.start()
        rdmas.append(r)

    acc = partial_chunk(my)
    for sl in (0, 2, 1):
        rdmas[slots.index(sl)].wait_recv()
        acc = acc + comm_ref[sl]
    out_ref[:, :] = acc
    for r in rdmas:
        r.wait_send()


def kernel(x, router_W, route_idx, expert_W):
    del router_W
    my = lax.axis_index("i")

    e = route_idx[:, 0].astype(jnp.int32)
    onehot = (e[:, None] == jnp.arange(E, dtype=jnp.int32)[None, :])
    cum = jnp.cumsum(onehot.astype(jnp.int32), axis=0)
    pos = jnp.sum(jnp.where(onehot, cum, 0), axis=1) - 1
    keep = pos < CAP
    el = e - E_LOCAL * my
    mine = (el >= 0) & (el < E_LOCAL) & keep
    slot = jnp.where(mine, el * CAP_PAD + pos, S).astype(jnp.int32)

    return pl.pallas_call(
        _fused_body,
        out_shape=jax.ShapeDtypeStruct((M, H), jnp.float32),
        in_specs=[
            pl.BlockSpec(memory_space=pltpu.VMEM),
            pl.BlockSpec(memory_space=pltpu.VMEM),
            pl.BlockSpec(memory_space=pltpu.VMEM),
            pl.BlockSpec(memory_space=pltpu.VMEM),
        ],
        out_specs=pl.BlockSpec(memory_space=pltpu.VMEM),
        scratch_shapes=[
            pltpu.VMEM((S, H), jnp.float32),
            pltpu.VMEM((N_DEV - 1, M, H), jnp.float32),
            pltpu.VMEM((N_DEV - 1, M, H), jnp.float32),
            pltpu.SemaphoreType.DMA((N_DEV - 1,)),
            pltpu.SemaphoreType.DMA((N_DEV - 1,)),
        ],
        compiler_params=pltpu.CompilerParams(collective_id=0),
    )(slot[None, :], slot[:, None], x, expert_W)


# device time: 48677 ns/iter; 2.3493x vs baseline; 1.5704x over previous
import jax
import jax.numpy as jnp
from jax import lax
from jax.experimental import pallas as pl
from jax.experimental.pallas import tpu as pltpu

N_DEV = 4
E = 32
E_LOCAL = 8
CAP = 51
CAP_PAD = 64
N_TOK = 2048
D = 512
H = 1024
M = N_TOK // N_DEV
S = E_LOCAL * CAP_PAD
CAPC = 192


def _fused_body(slotr_ref, slotc_ref, rkr_ref, rkc_ref, ownc_ref, x_ref,
                w_ref, out_ref, yg_ref, snd_ref, comm_ref,
                send_sems, recv_sems):
    my = lax.axis_index("i")
    left = jnp.mod(my - 1, N_DEV)
    right = jnp.mod(my + 1, N_DEV)
    diag = jnp.mod(my + 2, N_DEV)

    barrier_sem = pltpu.get_barrier_semaphore()
    for nbr in (left, right, diag):
        pl.semaphore_signal(
            barrier_sem, inc=1,
            device_id=(nbr,), device_id_type=pl.DeviceIdType.MESH,
        )
    pl.semaphore_wait(barrier_sem, 3)

    iota_s = lax.broadcasted_iota(jnp.int32, (S, N_TOK), 0)
    P = (iota_s == slotr_ref[:, :]).astype(jnp.float32)
    xg = jnp.dot(P, x_ref[:, :], preferred_element_type=jnp.float32)

    for e in range(E_LOCAL):
        yg_ref[e * CAP_PAD:(e + 1) * CAP_PAD, :] = jnp.dot(
            xg[e * CAP_PAD:(e + 1) * CAP_PAD, :], w_ref[e],
            preferred_element_type=jnp.float32,
        )
    yg = yg_ref[:, :]

    def compact_for(c):
        rk_c = rkr_ref[pl.ds(c, 1), :]
        rct = (lax.broadcasted_iota(jnp.int32, (CAPC, M), 0)
               == rk_c).astype(jnp.float32)
        slot_c = slotc_ref[pl.ds(c * M, M), :]
        pct = (lax.broadcasted_iota(jnp.int32, (M, S), 1)
               == slot_c).astype(jnp.float32)
        cc = jnp.dot(rct, pct, preferred_element_type=jnp.float32)
        return jnp.dot(cc, yg, preferred_element_type=jnp.float32)

    dsts = (diag, right, left)
    slots = (1, 0, 2)
    rdmas = []
    for (dst, sl) in zip(dsts, slots):
        snd_ref[sl] = compact_for(dst)
        r = pltpu.make_async_remote_copy(
            src_ref=snd_ref.at[sl], dst_ref=comm_ref.at[sl],
            send_sem=send_sems.at[sl], recv_sem=recv_sems.at[sl],
            device_id=(dst,), device_id_type=pl.DeviceIdType.MESH,
        )
        r.start()
        rdmas.append(r)

    slot_m = slotc_ref[pl.ds(my * M, M), :]
    pmt = (lax.broadcasted_iota(jnp.int32, (M, S), 1)
           == slot_m).astype(jnp.float32)
    acc = jnp.dot(pmt, yg, preferred_element_type=jnp.float32)

    rk_m = rkc_ref[pl.ds(my * M, M), :]
    own_m = ownc_ref[pl.ds(my * M, M), :]
    iota_c = lax.broadcasted_iota(jnp.int32, (M, CAPC), 1)
    for sl, k in ((0, 1), (2, 3), (1, 2)):
        src = jnp.mod(my - k, N_DEV)
        rdmas[slots.index(sl)].wait_recv()
        q = ((own_m == src) & (rk_m == iota_c)).astype(jnp.float32)
        acc = acc + jnp.dot(q, comm_ref[sl],
                            preferred_element_type=jnp.float32)
    out_ref[:, :] = acc
    for r in rdmas:
        r.wait_send()


def _tok_cumsum(a):
    k = a.shape[1]
    b = a.reshape(16, 128, k)
    within = jnp.cumsum(b, axis=1)
    totals = within[:, -1, :]
    prefix = jnp.cumsum(totals, axis=0) - totals
    return (within + prefix[:, None, :]).reshape(N_TOK, k)


def kernel(x, router_W, route_idx, expert_W):
    del router_W
    my = lax.axis_index("i")

    e = route_idx[:, 0].astype(jnp.int32)
    onehot = (e[:, None] == jnp.arange(E, dtype=jnp.int32)[None, :])
    cum = _tok_cumsum(onehot.astype(jnp.int32))
    pos = jnp.sum(jnp.where(onehot, cum, 0), axis=1) - 1
    keep = pos < CAP
    owner = e // E_LOCAL
    el = e - E_LOCAL * my
    mine = (owner == my) & keep
    slot = jnp.where(mine, el * CAP_PAD + pos, S).astype(jnp.int32)

    oh4 = (owner[:, None] == jnp.arange(N_DEV, dtype=jnp.int32)[None, :]) \
        & keep[:, None]
    cum4 = _tok_cumsum(oh4.astype(jnp.int32))
    cb = cum4.reshape(N_DEV, M, N_DEV)
    base = jnp.concatenate(
        [jnp.zeros((1, N_DEV), jnp.int32), cb[:-1, -1, :]], axis=0)
    blockcum = (cb - base[:, None, :]).reshape(N_TOK, N_DEV)
    rkv = (jnp.sum(jnp.where(oh4, blockcum, 0), axis=1) - 1).astype(jnp.int32)

    return pl.pallas_call(
        _fused_body,
        out_shape=jax.ShapeDtypeStruct((M, H), jnp.float32),
        in_specs=[pl.BlockSpec(memory_space=pltpu.VMEM)] * 7,
        out_specs=pl.BlockSpec(memory_space=pltpu.VMEM),
        scratch_shapes=[
            pltpu.VMEM((S, H), jnp.float32),
            pltpu.VMEM((N_DEV - 1, CAPC, H), jnp.float32),
            pltpu.VMEM((N_DEV - 1, CAPC, H), jnp.float32),
            pltpu.SemaphoreType.DMA((N_DEV - 1,)),
            pltpu.SemaphoreType.DMA((N_DEV - 1,)),
        ],
        compiler_params=pltpu.CompilerParams(collective_id=0),
    )(
        slot[None, :],
        slot[:, None],
        rkv.reshape(N_DEV, M),
        rkv[:, None],
        owner.astype(jnp.int32)[:, None],
        x, expert_W,
    )


# device time: 37949 ns/iter; 3.0134x vs baseline; 1.2827x over previous
import jax
import jax.numpy as jnp
from jax import lax
from jax.experimental import pallas as pl
from jax.experimental.pallas import tpu as pltpu

N_DEV = 4
E = 32
E_LOCAL = 8
CAP = 51
CAP_PAD = 64
N_TOK = 2048
D = 512
H = 1024
M = N_TOK // N_DEV
S = E_LOCAL * CAP_PAD
CAPC = 192

BF = jnp.bfloat16


def _fused_body(slotr_ref, rkr_ref, cols_ref, x_ref, w_ref,
                out_ref, yg_ref, snd_ref, comm_ref, send_sems, recv_sems):
    my = lax.axis_index("i")
    left = jnp.mod(my - 1, N_DEV)
    right = jnp.mod(my + 1, N_DEV)
    diag = jnp.mod(my + 2, N_DEV)

    barrier_sem = pltpu.get_barrier_semaphore()
    for nbr in (left, right, diag):
        pl.semaphore_signal(
            barrier_sem, inc=1,
            device_id=(nbr,), device_id_type=pl.DeviceIdType.MESH,
        )
    pl.semaphore_wait(barrier_sem, 3)

    iota_s = lax.broadcasted_iota(jnp.int32, (S, N_TOK), 0)
    P = (iota_s == slotr_ref[:, :]).astype(BF)
    xg = jnp.dot(P, x_ref[:, :],
                 preferred_element_type=jnp.float32).astype(BF)

    for e in range(E_LOCAL):
        yg_ref[e * CAP_PAD:(e + 1) * CAP_PAD, :] = jnp.dot(
            xg[e * CAP_PAD:(e + 1) * CAP_PAD, :], w_ref[e],
            preferred_element_type=jnp.float32,
        ).astype(BF)
    yg = yg_ref[:, :]

    def compact_for(c):
        rk_c = rkr_ref[pl.ds(c, 1), :]
        rct = (lax.broadcasted_iota(jnp.int32, (CAPC, M), 0)
               == rk_c).astype(BF)
        slot_c = cols_ref[pl.ds(c * M, M), 0:1]
        pct = (lax.broadcasted_iota(jnp.int32, (M, S), 1)
               == slot_c).astype(BF)
        cc = jnp.dot(rct, pct,
                     preferred_element_type=jnp.float32).astype(BF)
        return jnp.dot(cc, yg,
                       preferred_element_type=jnp.float32).astype(BF)

    dsts = (diag, right, left)
    slots = (1, 0, 2)
    rdmas = []
    for (dst, sl) in zip(dsts, slots):
        snd_ref[sl] = compact_for(dst)
        r = pltpu.make_async_remote_copy(
            src_ref=snd_ref.at[sl], dst_ref=comm_ref.at[sl],
            send_sem=send_sems.at[sl], recv_sem=recv_sems.at[sl],
            device_id=(dst,), device_id_type=pl.DeviceIdType.MESH,
        )
        r.start()
        rdmas.append(r)

    slot_m = cols_ref[pl.ds(my * M, M), 0:1]
    pmt = (lax.broadcasted_iota(jnp.int32, (M, S), 1) == slot_m).astype(BF)
    acc = jnp.dot(pmt, yg, preferred_element_type=jnp.float32)

    rk_m = cols_ref[pl.ds(my * M, M), 1:2]
    own_m = cols_ref[pl.ds(my * M, M), 2:3]
    iota_c = lax.broadcasted_iota(jnp.int32, (M, CAPC), 1)
    for sl, k in ((0, 1), (2, 3), (1, 2)):
        src = jnp.mod(my - k, N_DEV)
        rdmas[slots.index(sl)].wait_recv()
        q = ((own_m == src) & (rk_m == iota_c)).astype(BF)
        acc = acc + jnp.dot(q, comm_ref[sl],
                            preferred_element_type=jnp.float32)
    out_ref[:, :] = acc
    for r in rdmas:
        r.wait_send()


def _tok_cumsum(a):
    k = a.shape[1]
    b = a.reshape(16, 128, k)
    within = jnp.cumsum(b, axis=1)
    totals = within[:, -1, :]
    prefix = jnp.cumsum(totals, axis=0) - totals
    return (within + prefix[:, None, :]).reshape(N_TOK, k)


def kernel(x, router_W, route_idx, expert_W):
    del router_W
    my = lax.axis_index("i")

    e = route_idx[:, 0].astype(jnp.int32)
    onehot = (e[:, None] == jnp.arange(E, dtype=jnp.int32)[None, :])
    cum = _tok_cumsum(onehot.astype(jnp.int32))
    pos = jnp.sum(jnp.where(onehot, cum, 0), axis=1) - 1
    keep = pos < CAP
    owner = e // E_LOCAL
    el = e - E_LOCAL * my
    mine = (owner == my) & keep
    slot = jnp.where(mine, el * CAP_PAD + pos, S).astype(jnp.int32)

    oh4 = (owner[:, None] == jnp.arange(N_DEV, dtype=jnp.int32)[None, :]) \
        & keep[:, None]
    cum4 = _tok_cumsum(oh4.astype(jnp.int32))
    cb = cum4.reshape(N_DEV, M, N_DEV)
    base = jnp.concatenate(
        [jnp.zeros((1, N_DEV), jnp.int32), cb[:-1, -1, :]], axis=0)
    blockcum = (cb - base[:, None, :]).reshape(N_TOK, N_DEV)
    rkv = (jnp.sum(jnp.where(oh4, blockcum, 0), axis=1) - 1).astype(jnp.int32)

    cols = jnp.stack([slot, rkv, owner.astype(jnp.int32)], axis=1)

    return pl.pallas_call(
        _fused_body,
        out_shape=jax.ShapeDtypeStruct((M, H), jnp.float32),
        in_specs=[pl.BlockSpec(memory_space=pltpu.VMEM)] * 5,
        out_specs=pl.BlockSpec(memory_space=pltpu.VMEM),
        scratch_shapes=[
            pltpu.VMEM((S, H), BF),
            pltpu.VMEM((N_DEV - 1, CAPC, H), BF),
            pltpu.VMEM((N_DEV - 1, CAPC, H), BF),
            pltpu.SemaphoreType.DMA((N_DEV - 1,)),
            pltpu.SemaphoreType.DMA((N_DEV - 1,)),
        ],
        compiler_params=pltpu.CompilerParams(collective_id=0),
    )(
        slot[None, :],
        rkv.reshape(N_DEV, M),
        cols,
        x.astype(BF),
        expert_W.astype(BF),
    )


# device time: 36451 ns/iter; 3.1373x vs baseline; 1.0411x over previous
import jax
import jax.numpy as jnp
from jax import lax
from jax.experimental import pallas as pl
from jax.experimental.pallas import tpu as pltpu

N_DEV = 4
E = 32
E_LOCAL = 8
CAP = 51
CAP_PAD = 64
N_TOK = 2048
D = 512
H = 1024
M = N_TOK // N_DEV
S = E_LOCAL * CAP_PAD
CAPC = 192

BF = jnp.bfloat16

_DN_T = (((0,), (0,)), ((), ()))


def _fused_body(e_ref, cum_ref, bc_ref, x_ref, w_ref,
                out_ref, meta_ref, yg_ref, snd_ref, comm_ref,
                send_sems, recv_sems):
    my = lax.axis_index("i")
    left = jnp.mod(my - 1, N_DEV)
    right = jnp.mod(my + 1, N_DEV)
    diag = jnp.mod(my + 2, N_DEV)

    barrier_sem = pltpu.get_barrier_semaphore()
    for nbr in (left, right, diag):
        pl.semaphore_signal(
            barrier_sem, inc=1,
            device_id=(nbr,), device_id_type=pl.DeviceIdType.MESH,
        )
    pl.semaphore_wait(barrier_sem, 3)

    e_col = e_ref[:, :]
    onehot = e_col == lax.broadcasted_iota(jnp.int32, (N_TOK, E), 1)
    pos = jnp.sum(jnp.where(onehot, cum_ref[:, :], 0),
                  axis=1, keepdims=True) - 1
    keep = pos < CAP
    owner = e_col // E_LOCAL
    mine = (owner == my) & keep
    slot = jnp.where(mine, (e_col - E_LOCAL * my) * CAP_PAD + pos, S)
    oh4 = (owner == lax.broadcasted_iota(jnp.int32, (N_TOK, N_DEV), 1)) & keep
    rkv = jnp.sum(jnp.where(oh4, bc_ref[:, :], 0),
                  axis=1, keepdims=True) - 1
    meta_ref[:, 0:1] = slot
    meta_ref[:, 1:2] = rkv
    meta_ref[:, 2:3] = owner

    PT = (slot == lax.broadcasted_iota(jnp.int32, (N_TOK, S), 1)).astype(BF)
    xg = lax.dot_general(PT, x_ref[:, :], _DN_T,
                         preferred_element_type=jnp.float32)

    for e in range(E_LOCAL):
        yg_ref[e * CAP_PAD:(e + 1) * CAP_PAD, :] = jnp.dot(
            xg[e * CAP_PAD:(e + 1) * CAP_PAD, :], w_ref[e],
            preferred_element_type=jnp.float32,
        ).astype(BF)
    yg = yg_ref[:, :]

    def compact_for(c):
        rk_c = meta_ref[pl.ds(c * M, M), 1:2]
        rctT = (rk_c == lax.broadcasted_iota(jnp.int32, (M, CAPC), 1)
                ).astype(BF)
        slot_c = meta_ref[pl.ds(c * M, M), 0:1]
        pct = (slot_c == lax.broadcasted_iota(jnp.int32, (M, S), 1)
               ).astype(BF)
        cc = lax.dot_general(rctT, pct, _DN_T,
                             preferred_element_type=jnp.float32).astype(BF)
        return jnp.dot(cc, yg,
                       preferred_element_type=jnp.float32).astype(BF)

    dsts = (diag, right, left)
    slots = (1, 0, 2)
    rdmas = []
    for (dst, sl) in zip(dsts, slots):
        snd_ref[sl] = compact_for(dst)
        r = pltpu.make_async_remote_copy(
            src_ref=snd_ref.at[sl], dst_ref=comm_ref.at[sl],
            send_sem=send_sems.at[sl], recv_sem=recv_sems.at[sl],
            device_id=(dst,), device_id_type=pl.DeviceIdType.MESH,
        )
        r.start()
        rdmas.append(r)

    slot_m = meta_ref[pl.ds(my * M, M), 0:1]
    pmt = (slot_m == lax.broadcasted_iota(jnp.int32, (M, S), 1)).astype(BF)
    acc = jnp.dot(pmt, yg, preferred_element_type=jnp.float32)

    rk_m = meta_ref[pl.ds(my * M, M), 1:2]
    own_m = meta_ref[pl.ds(my * M, M), 2:3]
    iota_c = lax.broadcasted_iota(jnp.int32, (M, CAPC), 1)
    for sl, k in ((0, 1), (2, 3), (1, 2)):
        src = jnp.mod(my - k, N_DEV)
        rdmas[slots.index(sl)].wait_recv()
        q = ((own_m == src) & (rk_m == iota_c)).astype(BF)
        acc = acc + jnp.dot(q, comm_ref[sl],
                            preferred_element_type=jnp.float32)
    out_ref[:, :] = acc
    for r in rdmas:
        r.wait_send()


def _tok_cumsum(a):
    k = a.shape[1]
    b = a.reshape(16, 128, k)
    within = jnp.cumsum(b, axis=1)
    totals = within[:, -1, :]
    prefix = jnp.cumsum(totals, axis=0) - totals
    return (within + prefix[:, None, :]).reshape(N_TOK, k)


def kernel(x, router_W, route_idx, expert_W):
    del router_W
    e = route_idx[:, 0].astype(jnp.int32)

    onehot = (e[:, None] == jnp.arange(E, dtype=jnp.int32)[None, :])
    cum = _tok_cumsum(onehot.astype(jnp.int32))

    pos = jnp.sum(jnp.where(onehot, cum, 0), axis=1) - 1
    keep = pos < CAP
    owner = e // E_LOCAL
    oh4 = (owner[:, None] == jnp.arange(N_DEV, dtype=jnp.int32)[None, :]) \
        & keep[:, None]
    cum4 = _tok_cumsum(oh4.astype(jnp.int32))
    cb = cum4.reshape(N_DEV, M, N_DEV)
    base = jnp.concatenate(
        [jnp.zeros((1, N_DEV), jnp.int32), cb[:-1, -1, :]], axis=0)
    blockcum = (cb - base[:, None, :]).reshape(N_TOK, N_DEV)

    return pl.pallas_call(
        _fused_body,
        out_shape=jax.ShapeDtypeStruct((M, H), jnp.float32),
        in_specs=[pl.BlockSpec(memory_space=pltpu.VMEM)] * 5,
        out_specs=pl.BlockSpec(memory_space=pltpu.VMEM),
        scratch_shapes=[
            pltpu.VMEM((N_TOK, 3), jnp.int32),
            pltpu.VMEM((S, H), BF),
            pltpu.VMEM((N_DEV - 1, CAPC, H), BF),
            pltpu.VMEM((N_DEV - 1, CAPC, H), BF),
            pltpu.SemaphoreType.DMA((N_DEV - 1,)),
            pltpu.SemaphoreType.DMA((N_DEV - 1,)),
        ],
        compiler_params=pltpu.CompilerParams(collective_id=0),
    )(
        e[:, None],
        cum,
        blockcum,
        x.astype(BF),
        expert_W,
    )


# device time: 35617 ns/iter; 3.2107x vs baseline; 1.0234x over previous
import jax
import jax.numpy as jnp
from jax import lax
from jax.experimental import pallas as pl
from jax.experimental.pallas import tpu as pltpu

N_DEV = 4
E = 32
E_LOCAL = 8
CAP = 51
CAP_PAD = 64
N_TOK = 2048
D = 512
H = 1024
M = N_TOK // N_DEV
S = E_LOCAL * CAP_PAD
CAPC = 192

BF = jnp.bfloat16


def _fused_body(slotr_ref, rkr_ref, cols_ref, x_ref, w_ref,
                out_ref, yg_ref, snd_ref, comm_ref, send_sems, recv_sems):
    my = lax.axis_index("i")
    left = jnp.mod(my - 1, N_DEV)
    right = jnp.mod(my + 1, N_DEV)
    diag = jnp.mod(my + 2, N_DEV)

    barrier_sem = pltpu.get_barrier_semaphore()
    for nbr in (left, right, diag):
        pl.semaphore_signal(
            barrier_sem, inc=1,
            device_id=(nbr,), device_id_type=pl.DeviceIdType.MESH,
        )
    pl.semaphore_wait(barrier_sem, 3)

    iota_s = lax.broadcasted_iota(jnp.int32, (S, N_TOK), 0)
    P = (iota_s == slotr_ref[:, :]).astype(BF)
    xg = jnp.dot(P, x_ref[:, :], preferred_element_type=jnp.float32)

    for e in range(E_LOCAL):
        yg_ref[e * CAP_PAD:(e + 1) * CAP_PAD, :] = jnp.dot(
            xg[e * CAP_PAD:(e + 1) * CAP_PAD, :], w_ref[e],
            preferred_element_type=jnp.float32,
        ).astype(BF)
    yg = yg_ref[:, :]

    def compact_for(c):
        rk_c = rkr_ref[pl.ds(c, 1), :]
        rct = (lax.broadcasted_iota(jnp.int32, (CAPC, M), 0)
               == rk_c).astype(BF)
        slot_c = cols_ref[pl.ds(c * M, M), 0:1]
        pct = (lax.broadcasted_iota(jnp.int32, (M, S), 1)
               == slot_c).astype(BF)
        cc = jnp.dot(rct, pct,
                     preferred_element_type=jnp.float32).astype(BF)
        return jnp.dot(cc, yg,
                       preferred_element_type=jnp.float32).astype(BF)

    dsts = (diag, right, left)
    slots = (1, 0, 2)
    rdmas = []
    for (dst, sl) in zip(dsts, slots):
        snd_ref[sl] = compact_for(dst)
        r = pltpu.make_async_remote_copy(
            src_ref=snd_ref.at[sl], dst_ref=comm_ref.at[sl],
            send_sem=send_sems.at[sl], recv_sem=recv_sems.at[sl],
            device_id=(dst,), device_id_type=pl.DeviceIdType.MESH,
        )
        r.start()
        rdmas.append(r)

    slot_m = cols_ref[pl.ds(my * M, M), 0:1]
    pmt = (lax.broadcasted_iota(jnp.int32, (M, S), 1) == slot_m).astype(BF)
    acc = jnp.dot(pmt, yg, preferred_element_type=jnp.float32)

    rk_m = cols_ref[pl.ds(my * M, M), 1:2]
    own_m = cols_ref[pl.ds(my * M, M), 2:3]
    iota_c = lax.broadcasted_iota(jnp.int32, (M, CAPC), 1)
    for sl, k in ((0, 1), (2, 3), (1, 2)):
        src = jnp.mod(my - k, N_DEV)
        rdmas[slots.index(sl)].wait_recv()
        q = ((own_m == src) & (rk_m == iota_c)).astype(BF)
        acc = acc + jnp.dot(q, comm_ref[sl],
                            preferred_element_type=jnp.float32)
    out_ref[:, :] = acc
    for r in rdmas:
        r.wait_send()


def _tok_cumsum(a):
    k = a.shape[1]
    b = a.reshape(16, 128, k)
    within = jnp.cumsum(b, axis=1)
    totals = within[:, -1, :]
    prefix = jnp.cumsum(totals, axis=0) - totals
    return (within + prefix[:, None, :]).reshape(N_TOK, k)


def kernel(x, router_W, route_idx, expert_W):
    del router_W
    my = lax.axis_index("i")

    e = route_idx[:, 0].astype(jnp.int32)
    onehot = (e[:, None] == jnp.arange(E, dtype=jnp.int32)[None, :])
    cum = _tok_cumsum(onehot.astype(jnp.int32))
    pos = jnp.sum(jnp.where(onehot, cum, 0), axis=1) - 1
    keep = pos < CAP
    owner = e // E_LOCAL
    el = e - E_LOCAL * my
    mine = (owner == my) & keep
    slot = jnp.where(mine, el * CAP_PAD + pos, S).astype(jnp.int32)

    oh4 = (owner[:, None] == jnp.arange(N_DEV, dtype=jnp.int32)[None, :]) \
        & keep[:, None]
    cum4 = _tok_cumsum(oh4.astype(jnp.int32))
    cb = cum4.reshape(N_DEV, M, N_DEV)
    base = jnp.concatenate(
        [jnp.zeros((1, N_DEV), jnp.int32), cb[:-1, -1, :]], axis=0)
    blockcum = (cb - base[:, None, :]).reshape(N_TOK, N_DEV)
    rkv = (jnp.sum(jnp.where(oh4, blockcum, 0), axis=1) - 1).astype(jnp.int32)

    cols = jnp.stack([slot, rkv, owner.astype(jnp.int32)], axis=1)

    return pl.pallas_call(
        _fused_body,
        out_shape=jax.ShapeDtypeStruct((M, H), jnp.float32),
        in_specs=[pl.BlockSpec(memory_space=pltpu.VMEM)] * 5,
        out_specs=pl.BlockSpec(memory_space=pltpu.VMEM),
        scratch_shapes=[
            pltpu.VMEM((S, H), BF),
            pltpu.VMEM((N_DEV - 1, CAPC, H), BF),
            pltpu.VMEM((N_DEV - 1, CAPC, H), BF),
            pltpu.SemaphoreType.DMA((N_DEV - 1,)),
            pltpu.SemaphoreType.DMA((N_DEV - 1,)),
        ],
        compiler_params=pltpu.CompilerParams(collective_id=0),
    )(
        slot[None, :],
        rkv.reshape(N_DEV, M),
        cols,
        x.astype(BF),
        expert_W,
    )


# device time: 30611 ns/iter; 3.7358x vs baseline; 1.1635x over previous
import jax
import jax.numpy as jnp
from jax import lax
from jax.experimental import pallas as pl
from jax.experimental.pallas import tpu as pltpu

N_DEV = 4
E = 32
E_LOCAL = 8
CAP = 51
CAP_PAD = 64
N_TOK = 2048
D = 512
H = 1024
M = N_TOK // N_DEV
S = E_LOCAL * CAP_PAD
CAPC = 192

BF = jnp.bfloat16


def _fused_body(slotr_ref, rkr_ref, cols_ref, x_ref, w_ref,
                out_ref, yg_ref, snd_ref, comm_ref, send_sems, recv_sems):
    my = lax.axis_index("i")
    left = jnp.mod(my - 1, N_DEV)
    right = jnp.mod(my + 1, N_DEV)
    diag = jnp.mod(my + 2, N_DEV)

    barrier_sem = pltpu.get_barrier_semaphore()
    for nbr in (left, right, diag):
        pl.semaphore_signal(
            barrier_sem, inc=1,
            device_id=(nbr,), device_id_type=pl.DeviceIdType.MESH,
        )
    pl.semaphore_wait(barrier_sem, 3)

    iota_s = lax.broadcasted_iota(jnp.int32, (S, N_TOK), 0)
    P = (iota_s == slotr_ref[:, :]).astype(BF)
    xg = jnp.dot(P, x_ref[:, :], preferred_element_type=jnp.float32)

    for e in range(E_LOCAL):
        yg_ref[e * CAP_PAD:(e + 1) * CAP_PAD, :] = jnp.dot(
            xg[e * CAP_PAD:(e + 1) * CAP_PAD, :], w_ref[e],
            preferred_element_type=jnp.float32,
        ).astype(BF)
    yg = yg_ref[:, :]

    def compact_for(c):
        rk_c = rkr_ref[pl.ds(c, 1), :]
        rct = (lax.broadcasted_iota(jnp.int32, (CAPC, M), 0)
               == rk_c).astype(BF)
        slot_c = cols_ref[pl.ds(c * M, M), 0:1]
        pct = (lax.broadcasted_iota(jnp.int32, (M, S), 1)
               == slot_c).astype(BF)
        cc = jnp.dot(rct, pct,
                     preferred_element_type=jnp.float32).astype(BF)
        return jnp.dot(cc, yg,
                       preferred_element_type=jnp.float32).astype(BF)

    dsts = (diag, right, left)
    slots = (1, 0, 2)
    rdmas = []
    for (dst, sl) in zip(dsts, slots):
        snd_ref[sl] = compact_for(dst)
        r = pltpu.make_async_remote_copy(
            src_ref=snd_ref.at[sl], dst_ref=comm_ref.at[sl],
            send_sem=send_sems.at[sl], recv_sem=recv_sems.at[sl],
            device_id=(dst,), device_id_type=pl.DeviceIdType.MESH,
        )
        r.start()
        rdmas.append(r)

    slot_m = cols_ref[pl.ds(my * M, M), 0:1]
    pmt = (lax.broadcasted_iota(jnp.int32, (M, S), 1) == slot_m).astype(BF)
    acc = jnp.dot(pmt, yg, preferred_element_type=jnp.float32)

    rk_m = cols_ref[pl.ds(my * M, M), 1:2]
    own_m = cols_ref[pl.ds(my * M, M), 2:3]
    iota_c = lax.broadcasted_iota(jnp.int32, (M, CAPC), 1)
    for sl, k in ((0, 1), (2, 3), (1, 2)):
        src = jnp.mod(my - k, N_DEV)
        rdmas[slots.index(sl)].wait_recv()
        q = ((own_m == src) & (rk_m == iota_c)).astype(BF)
        acc = acc + jnp.dot(q, comm_ref[sl],
                            preferred_element_type=jnp.float32)
    out_ref[:, :] = acc
    for r in rdmas:
        r.wait_send()


def _tok_cumsum(a):
    k = a.shape[1]
    b = a.reshape(16, 128, k)
    within = jnp.cumsum(b, axis=1)
    totals = within[:, -1, :]
    prefix = jnp.cumsum(totals, axis=0) - totals
    return (within + prefix[:, None, :]).reshape(N_TOK, k)


def kernel(x, router_W, route_idx, expert_W):
    del router_W
    my = lax.axis_index("i")

    e = route_idx[:, 0].astype(jnp.int32)
    onehot = (e[:, None] == jnp.arange(E, dtype=jnp.int32)[None, :])
    cum = _tok_cumsum(onehot.astype(jnp.int32))
    pos = cum[:, E - 1]
    for j in range(E - 1):
        pos = jnp.where(e == j, cum[:, j], pos)
    pos = pos - 1
    keep = pos < CAP
    owner = e // E_LOCAL
    el = e - E_LOCAL * my
    mine = (owner == my) & keep
    slot = jnp.where(mine, el * CAP_PAD + pos, S).astype(jnp.int32)

    oh4 = (owner[:, None] == jnp.arange(N_DEV, dtype=jnp.int32)[None, :]) \
        & keep[:, None]
    cum4 = _tok_cumsum(oh4.astype(jnp.int32))
    cb = cum4.reshape(N_DEV, M, N_DEV)
    base = jnp.concatenate(
        [jnp.zeros((1, N_DEV), jnp.int32), cb[:-1, -1, :]], axis=0)
    blockcum = (cb - base[:, None, :]).reshape(N_TOK, N_DEV)
    rkv = blockcum[:, N_DEV - 1]
    for j in range(N_DEV - 1):
        rkv = jnp.where(owner == j, blockcum[:, j], rkv)
    rkv = jnp.where(keep, rkv - 1, -1).astype(jnp.int32)

    cols = jnp.stack([slot, rkv, owner.astype(jnp.int32)], axis=1)

    return pl.pallas_call(
        _fused_body,
        out_shape=jax.ShapeDtypeStruct((M, H), jnp.float32),
        in_specs=[pl.BlockSpec(memory_space=pltpu.VMEM)] * 5,
        out_specs=pl.BlockSpec(memory_space=pltpu.VMEM),
        scratch_shapes=[
            pltpu.VMEM((S, H), BF),
            pltpu.VMEM((N_DEV - 1, CAPC, H), BF),
            pltpu.VMEM((N_DEV - 1, CAPC, H), BF),
            pltpu.SemaphoreType.DMA((N_DEV - 1,)),
            pltpu.SemaphoreType.DMA((N_DEV - 1,)),
        ],
        compiler_params=pltpu.CompilerParams(collective_id=0),
    )(
        slot[None, :],
        rkv.reshape(N_DEV, M),
        cols,
        x.astype(BF),
        expert_W,
    )
